# baseline (speedup 1.0000x reference)
import jax
import jax.numpy as jnp
from jax.experimental import pallas as pl
from jax.experimental.pallas import tpu as pltpu
from jax.experimental.pallas import tpu_sc as plsc

B, N, C, W, H = 8, 16, 64, 32, 32
TS = 64


def _tv_build_kernel(t_ref, emb_ref, tv_ref):
    for n in range(N):
        row = emb_ref[t_ref[n], :]
        tv_ref[n] = jax.lax.broadcast_in_dim(row, (TS, W, H), (0,))


def _build_tv(t, emb_table):
    return pl.pallas_call(
        _tv_build_kernel,
        in_specs=[
            pl.BlockSpec(memory_space=pltpu.SMEM),
            pl.BlockSpec(memory_space=pltpu.VMEM),
        ],
        out_specs=pl.BlockSpec(memory_space=pltpu.VMEM),
        out_shape=jax.ShapeDtypeStruct((N, TS, W, H), emb_table.dtype),
    )(t, emb_table)


def kernel(x, t, emb_table):
    tv = _build_tv(t, emb_table)

    @pl.kernel(
        out_type=jax.ShapeDtypeStruct((B, N, C + TS, W, H), x.dtype),
        mesh=plsc.ScalarSubcoreMesh(axis_name="core", num_cores=2),
        scratch_types=[pltpu.SemaphoreType.DMA],
    )
    def sc_assemble(x_hbm, tv_hbm, o_hbm, sem):
        core = jax.lax.axis_index("core")

        copies = []
        for b in range(B):
            for n in range(N):
                if (b * N + n) % 2 == 0:
                    par = 0
                else:
                    par = 1
                copies.append((par, pltpu.make_async_copy(
                    x_hbm.at[b, n], o_hbm.at[b, n, 0:C], sem)))
                copies.append((par, pltpu.make_async_copy(
                    tv_hbm.at[n], o_hbm.at[b, n, C:], sem)))

        @pl.when(core == 0)
        def _():
            for par, cp in copies:
                if par == 0:
                    cp.start()
            for par, cp in copies:
                if par == 0:
                    cp.wait()

        @pl.when(core == 1)
        def _():
            for par, cp in copies:
                if par == 1:
                    cp.start()
            for par, cp in copies:
                if par == 1:
                    cp.wait()

    return sc_assemble(x, tv)


# speedup vs baseline: 59.7924x; 59.7924x over previous
import jax
import jax.numpy as jnp
from jax.experimental import pallas as pl
from jax.experimental.pallas import tpu as pltpu

B, N, C, W, H = 8, 16, 64, 32, 32
WH = W * H
TS = 64
CH = 32
ROWS = (B * N) // CH
NBUF = 16
NGRP = N // ROWS


def _assemble_kernel(x_ref, t_ref, emb_ref, out_ref, stage_ref, insem, outsem):
    for k in range(NBUF):
        for r in range(ROWS):
            n = (k % NGRP) * ROWS + r
            row = emb_ref[t_ref[n], :]
            stage_ref[k, r, C:, :] = jax.lax.broadcast_in_dim(row, (TS, WH), (0,))

    def in_copy(c):
        k = c % NBUF
        cp = pltpu.make_async_copy(
            x_ref.at[c], stage_ref.at[k, :, 0:C, :], insem.at[k])
        cp.start(priority=c % 2)
        return cp

    def out_copy(c):
        k = c % NBUF
        cp = pltpu.make_async_copy(stage_ref.at[k], out_ref.at[c], outsem.at[k])
        cp.start(priority=c % 2)
        return cp

    ins = {}
    outs = {}
    for c in range(NBUF):
        ins[c] = in_copy(c)
    for c in range(NBUF):
        ins[c].wait()
        outs[c] = out_copy(c)
    for c in range(NBUF, CH):
        outs[c - NBUF].wait()
        ins[c] = in_copy(c)
    for c in range(NBUF, CH):
        ins[c].wait()
        outs[c] = out_copy(c)
    for c in range(NBUF, CH):
        outs[c].wait()


def kernel(x, t, emb_table):
    x4 = x.reshape(CH, ROWS, C, WH)
    out = pl.pallas_call(
        _assemble_kernel,
        in_specs=[
            pl.BlockSpec(memory_space=pl.ANY),
            pl.BlockSpec(memory_space=pltpu.SMEM),
            pl.BlockSpec(memory_space=pltpu.VMEM),
        ],
        out_specs=pl.BlockSpec(memory_space=pl.ANY),
        out_shape=jax.ShapeDtypeStruct((CH, ROWS, C + TS, WH), x.dtype),
        scratch_shapes=[
            pltpu.VMEM((NBUF, ROWS, C + TS, WH), x.dtype),
            pltpu.SemaphoreType.DMA((NBUF,)),
            pltpu.SemaphoreType.DMA((NBUF,)),
        ],
    )(x4, t.astype(jnp.int32), emb_table)
    return out.reshape(B, N, C + TS, W, H)
